# baseline (speedup 1.0000x reference)
import functools

import jax
import jax.numpy as jnp
from jax import lax
from jax.experimental import pallas as pl
from jax.experimental.pallas import tpu as pltpu
from jax.experimental.pallas import tpu_sc as plsc

_B = 128
_INPUT = 1024
_M = 4096
_W = 64
_R = 4
_C = 33
_IFACE = 323

_NC = 2
_NS = 16
_NW = _NC * _NS

_NIDX = _B * _C
_PER_W = ((_NIDX + _NW - 1) // _NW + 7) // 8 * 8
_NPAD = _PER_W * _NW


_HALF = _NPAD // 2
_CHUNK = 128


def _sc_gather(table, idx):
    mesh = plsc.VectorSubcoreMesh(core_axis_name="c", subcore_axis_name="s")

    @functools.partial(
        pl.kernel,
        mesh=mesh,
        out_type=jax.ShapeDtypeStruct((_NPAD, 2 * _W), jnp.float32),
        scratch_types=[
            pltpu.VMEM((_PER_W,), jnp.int32),
            pltpu.VMEM((_PER_W, 2 * _W), jnp.float32),
            pltpu.SemaphoreType.DMA,
        ],
    )
    def gather_kernel(table_hbm, idx_hbm, out_hbm, idx_v, rows_v, sem):
        wid = lax.axis_index("s") * _NC + lax.axis_index("c")
        base = wid * _PER_W
        pltpu.sync_copy(idx_hbm.at[pl.ds(base, _PER_W)], idx_v)
        for off, sz in ((0, 72), (72, 64)):
            pltpu.async_copy(table_hbm.at[idx_v.at[pl.ds(off, sz)]],
                             rows_v.at[pl.ds(off, sz)], sem).wait()
        pltpu.sync_copy(rows_v, out_hbm.at[pl.ds(base, _PER_W)])

    return gather_kernel(table, idx)


def _tc_iface(x_ref, w_ref, b_ref, lnw_ref, lnb_ref, rest_ref, st_ref):
    x = x_ref[...]
    wint = w_ref[...]
    xi = lax.dot_general(
        x, wint, (((1,), (1,)), ((), ())),
        precision=lax.Precision.HIGHEST,
    ) + b_ref[...]

    st_ref[...] = 1.0 + jax.nn.softplus(xi[:, _IFACE - 1:_IFACE])

    rest = xi[:, :_IFACE - 1]
    u = jnp.mean(rest, axis=-1, keepdims=True)
    s = jnp.mean((rest - u) ** 2, axis=-1, keepdims=True)
    rest_ref[...] = (lnw_ref[...] * (rest - u) / jnp.sqrt(s + 1e-12)
                     + lnb_ref[...])


def _tc_body(rest_ref, st_ref, pair_ref, par_ref, out_ref):
    rest = rest_ref[...]
    strength = st_ref[...]
    rw_end = _R * _W
    wv = rest[:, rw_end:rw_end + _W][:, None, :]
    ig = jax.nn.sigmoid(rest[:, rw_end + _W:rw_end + _W + 1])
    wg = jax.nn.sigmoid(rest[:, rw_end + _W + 1:rw_end + _W + 2])

    pairs = pair_ref[...]
    vis = jnp.where(par_ref[...] > 0.5,
                    pairs[:, :, _W:], pairs[:, :, :_W])
    wvn = jnp.sqrt(jnp.sum(wv * wv, axis=-1, keepdims=True))
    visn = jnp.sqrt(jnp.sum(vis * vis, axis=-1, keepdims=True))
    cw = jnp.sum(vis * wv, axis=-1, keepdims=True) / (wvn * visn + 1e-6)

    t = cw * (1.0 + ig)[:, :, None] * 5.0
    m = jnp.max(t, axis=1, keepdims=True)
    e = jnp.exp(t - m)
    ww = e / jnp.sum(e, axis=1, keepdims=True) * wg[:, :, None]

    nv = vis * (1.0 - ww) + ww * wv
    nvn = jnp.sqrt(jnp.sum(nv * nv, axis=-1, keepdims=True))

    out_ref[:, _R:, :] = nv

    st = strength[:, :, None]
    for r in range(_R):
        rq = rest[:, r * _W:(r + 1) * _W][:, None, :]
        rqn = jnp.sqrt(jnp.sum(rq * rq, axis=-1, keepdims=True))
        cr = jnp.sum(nv * rq, axis=-1, keepdims=True) / (rqn * nvn + 1e-6)
        tr = cr * st
        mr = jnp.max(tr, axis=1, keepdims=True)
        er = jnp.exp(tr - mr)
        rw = er / jnp.sum(er, axis=1, keepdims=True)
        out_ref[:, r, :] = jnp.sum(rw * nv, axis=1)


def kernel(x, memory, read_positions, W_int, b_int, ln_w, ln_b):
    b, m, w = memory.shape
    c = read_positions.shape[1]

    table = memory.reshape(b * m // 2, 2 * w)
    flat_idx = (read_positions
                + (jnp.arange(b, dtype=jnp.int32) * m)[:, None])
    pair_idx = jnp.concatenate(
        [(flat_idx >> 1).reshape(-1),
         jnp.zeros((_NPAD - _NIDX,), dtype=jnp.int32)])
    parity = (flat_idx & 1).astype(jnp.float32)[:, :, None]
    rows = _sc_gather(table, pair_idx)
    vis = rows[:_NIDX].reshape(b, c, 2 * w)

    rest, strength = pl.pallas_call(
        _tc_iface,
        out_shape=(jax.ShapeDtypeStruct((b, _IFACE - 1), jnp.float32),
                   jax.ShapeDtypeStruct((b, 1), jnp.float32)),
    )(x, W_int, b_int.reshape(1, -1), ln_w.reshape(1, -1),
      ln_b.reshape(1, -1))

    out = pl.pallas_call(
        _tc_body,
        out_shape=jax.ShapeDtypeStruct((b, _R + c, w), jnp.float32),
    )(rest, strength, vis, parity)
    return out


# speedup vs baseline: 1.7424x; 1.7424x over previous
import functools

import jax
import jax.numpy as jnp
from jax import lax
from jax.experimental import pallas as pl
from jax.experimental.pallas import tpu as pltpu
from jax.experimental.pallas import tpu_sc as plsc

_B = 128
_INPUT = 1024
_M = 4096
_W = 64
_R = 4
_C = 33
_IFACE = 323

_NC = 2
_NS = 16
_NW = _NC * _NS

_NIDX = _B * _C
_PER_W = ((_NIDX + _NW - 1) // _NW + 7) // 8 * 8
_NPAD = _PER_W * _NW


_HALF = _NPAD // 2
_CHUNK = 128


def _sc_gather(table, idx):
    mesh = plsc.ScalarSubcoreMesh(axis_name="core", num_cores=_NC)
    nchunks = _HALF // _CHUNK

    @functools.partial(
        pl.kernel,
        mesh=mesh,
        out_type=jax.ShapeDtypeStruct((_NPAD, _W), jnp.float32),
        scratch_types=[
            pltpu.SMEM((2, _CHUNK), jnp.int32),
            pltpu.SemaphoreType.DMA,
            pltpu.SemaphoreType.DMA,
            pltpu.SemaphoreType.DMA,
        ],
    )
    def gather_kernel(table_hbm, idx_hbm, out_hbm, idx_s, isem0, isem1, sem):
        base = lax.axis_index("core") * _HALF
        isems = (isem0, isem1)
        pltpu.async_copy(
            idx_hbm.at[pl.ds(base, _CHUNK)], idx_s.at[0], isem0).start()

        for ci in range(nchunks):
            j = ci * _CHUNK
            cur = ci % 2
            if ci + 1 < nchunks:
                pltpu.async_copy(
                    idx_hbm.at[pl.ds(base + j + _CHUNK, _CHUNK)],
                    idx_s.at[1 - cur], isems[1 - cur]).start()
            pltpu.make_async_copy(
                idx_hbm.at[pl.ds(base + j, _CHUNK)],
                idx_s.at[cur], isems[cur]).wait()

            @pl.loop(0, _CHUNK, step=4)
            def _(i, j=j, cur=cur):
                for u in range(4):
                    pltpu.make_async_copy(
                        table_hbm.at[idx_s[cur, i + u]],
                        out_hbm.at[base + j + i + u], sem).start()

        pltpu.make_async_copy(
            out_hbm.at[pl.ds(base, _HALF)],
            out_hbm.at[pl.ds(base, _HALF)], sem).wait()

    return gather_kernel(table, idx)


def _tc_iface(x_ref, w_ref, b_ref, lnw_ref, lnb_ref, rest_ref, st_ref):
    x = x_ref[...]
    wint = w_ref[...]
    xi = lax.dot_general(
        x, wint, (((1,), (1,)), ((), ())),
        precision=lax.Precision.HIGHEST,
    ) + b_ref[...]

    st_ref[...] = 1.0 + jax.nn.softplus(xi[:, _IFACE - 1:_IFACE])

    rest = xi[:, :_IFACE - 1]
    u = jnp.mean(rest, axis=-1, keepdims=True)
    s = jnp.mean((rest - u) ** 2, axis=-1, keepdims=True)
    rest_ref[...] = (lnw_ref[...] * (rest - u) / jnp.sqrt(s + 1e-12)
                     + lnb_ref[...])


def _tc_body(rest_ref, st_ref, vis_ref, out_ref):
    rest = rest_ref[...]
    strength = st_ref[...]
    rw_end = _R * _W
    wv = rest[:, rw_end:rw_end + _W][:, None, :]
    ig = jax.nn.sigmoid(rest[:, rw_end + _W:rw_end + _W + 1])
    wg = jax.nn.sigmoid(rest[:, rw_end + _W + 1:rw_end + _W + 2])

    vis = vis_ref[...]
    wvn = jnp.sqrt(jnp.sum(wv * wv, axis=-1, keepdims=True))
    visn = jnp.sqrt(jnp.sum(vis * vis, axis=-1, keepdims=True))
    cw = jnp.sum(vis * wv, axis=-1, keepdims=True) / (wvn * visn + 1e-6)

    t = cw * (1.0 + ig)[:, :, None] * 5.0
    m = jnp.max(t, axis=1, keepdims=True)
    e = jnp.exp(t - m)
    ww = e / jnp.sum(e, axis=1, keepdims=True) * wg[:, :, None]

    nv = vis * (1.0 - ww) + ww * wv
    nvn = jnp.sqrt(jnp.sum(nv * nv, axis=-1, keepdims=True))

    out_ref[:, _R:, :] = nv

    st = strength[:, :, None]
    for r in range(_R):
        rq = rest[:, r * _W:(r + 1) * _W][:, None, :]
        rqn = jnp.sqrt(jnp.sum(rq * rq, axis=-1, keepdims=True))
        cr = jnp.sum(nv * rq, axis=-1, keepdims=True) / (rqn * nvn + 1e-6)
        tr = cr * st
        mr = jnp.max(tr, axis=1, keepdims=True)
        er = jnp.exp(tr - mr)
        rw = er / jnp.sum(er, axis=1, keepdims=True)
        out_ref[:, r, :] = jnp.sum(rw * nv, axis=1)


def kernel(x, memory, read_positions, W_int, b_int, ln_w, ln_b):
    b, m, w = memory.shape
    c = read_positions.shape[1]

    table = memory.reshape(b * m, w)
    flat_idx = (read_positions
                + (jnp.arange(b, dtype=jnp.int32) * m)[:, None])
    flat_idx = jnp.concatenate(
        [flat_idx.reshape(-1),
         jnp.zeros((_NPAD - _NIDX,), dtype=jnp.int32)])
    rows = _sc_gather(table, flat_idx)
    vis = rows[:_NIDX].reshape(b, c, w)

    rest, strength = pl.pallas_call(
        _tc_iface,
        out_shape=(jax.ShapeDtypeStruct((b, _IFACE - 1), jnp.float32),
                   jax.ShapeDtypeStruct((b, 1), jnp.float32)),
    )(x, W_int, b_int.reshape(1, -1), ln_w.reshape(1, -1),
      ln_b.reshape(1, -1))

    out = pl.pallas_call(
        _tc_body,
        out_shape=jax.ShapeDtypeStruct((b, _R + c, w), jnp.float32),
    )(rest, strength, vis)
    return out
